# Initial kernel scaffold; baseline (speedup 1.0000x reference)
import jax
import jax.numpy as jnp
from jax.experimental import pallas as pl


def kernel(x, edge_index, W1, b1, W2, b2):
    raise NotImplementedError("write your pallas kernel here")


# speedup vs baseline: 10.0811x; 10.0811x over previous
import functools

import jax
import jax.numpy as jnp
from jax import lax
from jax.experimental import pallas as pl
from jax.experimental.pallas import tpu as pltpu
from jax.experimental.pallas import tpu_sc as plsc

N = 10000
D_IN = 128
D_H = 256
D_OUT = 128
E = 320000

N_PAD = 10240
PAD_NODE = N
E_ROWS = 2560
E_PAD = E_ROWS * 128

NC = 2
NS = 16
ROWS_W = E_ROWS // (NC * NS)
ROWS_T = E_ROWS // NS
ROWS_SC = 16
NSC = ROWS_T // ROWS_SC
ROWS_OUT = N_PAD // NS

_mesh = plsc.VectorSubcoreMesh(core_axis_name="c", subcore_axis_name="s")
_sc_params = pltpu.CompilerParams(needs_layout_passes=False)


@functools.partial(
    pl.kernel,
    out_type=jax.ShapeDtypeStruct((NC * NS, N_PAD), jnp.float32),
    mesh=_mesh,
    compiler_params=_sc_params,
    scratch_types=[
        pltpu.VMEM((ROWS_W, 128), jnp.int32),
        pltpu.VMEM((N_PAD,), jnp.float32),
    ],
)
def _deg_kernel(dst_hbm, out_hbm, idx_v, hist_v):
    c = lax.axis_index("c")
    s = lax.axis_index("s")
    w = c * NS + s

    zero16 = jnp.zeros((16,), jnp.float32)

    def zbody(i, carry):
        hist_v[pl.ds(i * 16, 16)] = zero16
        return carry

    lax.fori_loop(0, N_PAD // 16, zbody, 0)

    pltpu.sync_copy(dst_hbm.at[pl.ds(w * ROWS_W, ROWS_W)], idx_v)

    ones16 = jnp.ones((16,), jnp.float32)

    def ebody(i, carry):
        r = i // 8
        j = i % 8
        iv = idx_v[r, pl.ds(j * 16, 16)]
        plsc.addupdate_scatter(hist_v, [iv], ones16)
        return carry

    lax.fori_loop(0, ROWS_W * 8, ebody, 0)

    pltpu.sync_copy(hist_v, out_hbm.at[w])


def _make_scatter(edge_split):
    D = 128
    rows_t = E_ROWS // (NC * NS) if edge_split else E_ROWS // NS
    nsc = rows_t // ROWS_SC

    @functools.partial(
        pl.kernel,
        out_type=(
            jax.ShapeDtypeStruct((N_PAD, D), jnp.float32),
            jax.ShapeDtypeStruct((N_PAD, D), jnp.float32),
        ),
        mesh=_mesh,
        compiler_params=_sc_params,
        scratch_types=[
            pltpu.VMEM((ROWS_SC, 128), jnp.int32),
            pltpu.VMEM((ROWS_SC, 128), jnp.int32),
            pltpu.VMEM((2, 128, D), jnp.float32),
            pltpu.VMEM_SHARED((N_PAD, D), jnp.float32),
            pltpu.SemaphoreType.DMA,
            pltpu.SemaphoreType.DMA,
            pltpu.SemaphoreType.DMA,
            pltpu.SemaphoreType.DMA,
        ],
    )
    def scat(ylo, yhi, src_hbm, dst_hbm, olo, ohi,
             src_v, dst_v, bufs, acc_sh, g0, g1, s0, s1):
        c = lax.axis_index("c")
        s = lax.axis_index("s")
        gsems = (g0, g1)
        ssems = (s0, s1)

        rb = (c * NS + s) * rows_t if edge_split else s * rows_t

        zero16 = jnp.zeros((16,), jnp.float32)

        def zbody(i, carry):
            r = i // (D // 16)
            j = i % (D // 16)
            bufs[0, r, pl.ds(j * 16, 16)] = zero16
            return carry

        lax.fori_loop(0, 128 * (D // 16), zbody, 0)
        ob = s * ROWS_OUT
        for k in range(ROWS_OUT // 128):
            pltpu.sync_copy(bufs.at[0], acc_sh.at[pl.ds(ob + k * 128, 128)])
        plsc.subcore_barrier()

        def body(ytab, otab):
            def fire_gather(b, j):
                pltpu.async_copy(ytab.at[src_v.at[b]], bufs.at[j], gsems[j])

            def wait_gather(b, j):
                pltpu.make_async_copy(
                    ytab.at[src_v.at[b]], bufs.at[j], gsems[j]).wait()

            def fire_scatter(b, j):
                pltpu.async_copy(
                    bufs.at[j], acc_sh.at[dst_v.at[b]], ssems[j], add=True)

            def wait_scatter(b, j):
                pltpu.make_async_copy(
                    bufs.at[j], acc_sh.at[dst_v.at[b]], ssems[j]).wait()

            def chunk_body(ci, carry):
                rbase = rb + ci * ROWS_SC
                pltpu.sync_copy(src_hbm.at[pl.ds(rbase, ROWS_SC)], src_v)
                pltpu.sync_copy(dst_hbm.at[pl.ds(rbase, ROWS_SC)], dst_v)
                fire_gather(0, 0)
                for b in range(ROWS_SC):
                    j = b % 2
                    wait_gather(b, j)
                    fire_scatter(b, j)
                    if b > 0:
                        wait_scatter(b - 1, 1 - j)
                    if b + 1 < ROWS_SC:
                        fire_gather(b + 1, 1 - j)
                wait_scatter(ROWS_SC - 1, (ROWS_SC - 1) % 2)
                return carry

            lax.fori_loop(0, nsc, chunk_body, 0)

            plsc.subcore_barrier()
            for k in range(ROWS_OUT // 128):
                pltpu.sync_copy(acc_sh.at[pl.ds(ob + k * 128, 128)],
                                otab.at[pl.ds(ob + k * 128, 128)])

        @pl.when(c == 0)
        def _():
            body(ylo, olo)

        @pl.when(c == 1)
        def _():
            body(ylo if edge_split else yhi, ohi)

    return scat


_scatter_h = _make_scatter(edge_split=False)
_scatter_o = _make_scatter(edge_split=True)


_BLK = 512
_GRID = N_PAD // _BLK


def _dis_block(pt):
    deg = jnp.sum(pt, axis=1, keepdims=True) + 1.0
    return lax.rsqrt(deg)


def _tc1_body(x_ref, w_ref, pt_ref, ylo_ref, yhi_ref):
    dis = _dis_block(pt_ref[...])
    y = jnp.dot(x_ref[...], w_ref[...],
                preferred_element_type=jnp.float32) * dis
    ylo_ref[...] = y[:, :D_H // 2]
    yhi_ref[...] = y[:, D_H // 2:]


def _tc2_body(alo_ref, ahi_ref, ylo_ref, yhi_ref, pt_ref, b1_ref, w2_ref,
              y2_ref):
    dis = _dis_block(pt_ref[...])
    pre = jnp.concatenate(
        [alo_ref[...] + ylo_ref[...], ahi_ref[...] + yhi_ref[...]], axis=1)
    h = jnp.maximum(pre * dis + b1_ref[...], 0.0)
    y2_ref[...] = jnp.dot(h, w2_ref[...],
                          preferred_element_type=jnp.float32) * dis


def _tc3_body(a0_ref, a1_ref, y2_ref, pt_ref, b2_ref, out_ref):
    dis = _dis_block(pt_ref[...])
    pre = a0_ref[...] + a1_ref[...] + y2_ref[...]
    out_ref[...] = jnp.maximum(pre * dis + b2_ref[...], 0.0)


def _row_spec(d):
    return pl.BlockSpec((_BLK, d), lambda i: (i, 0))


def _full_spec(r, d):
    return pl.BlockSpec((r, d), lambda i: (0, 0))


_tc1 = pl.pallas_call(
    _tc1_body,
    grid=(_GRID,),
    in_specs=[_row_spec(D_IN), _full_spec(D_IN, D_H), _row_spec(NC * NS)],
    out_specs=(_row_spec(D_H // 2), _row_spec(D_H // 2)),
    out_shape=(jax.ShapeDtypeStruct((N_PAD, D_H // 2), jnp.float32),
               jax.ShapeDtypeStruct((N_PAD, D_H // 2), jnp.float32)),
)

_tc2 = pl.pallas_call(
    _tc2_body,
    grid=(_GRID,),
    in_specs=[_row_spec(D_H // 2), _row_spec(D_H // 2),
              _row_spec(D_H // 2), _row_spec(D_H // 2),
              _row_spec(NC * NS), _full_spec(1, D_H), _full_spec(D_H, D_OUT)],
    out_specs=_row_spec(D_OUT),
    out_shape=jax.ShapeDtypeStruct((N_PAD, D_OUT), jnp.float32),
)

_tc3 = pl.pallas_call(
    _tc3_body,
    grid=(_GRID,),
    in_specs=[_row_spec(D_OUT), _row_spec(D_OUT), _row_spec(D_OUT),
              _row_spec(NC * NS), _full_spec(1, D_OUT)],
    out_specs=_row_spec(D_OUT),
    out_shape=jax.ShapeDtypeStruct((N_PAD, D_OUT), jnp.float32),
)


def kernel(x, edge_index, W1, b1, W2, b2):
    ei = edge_index.astype(jnp.int32)
    pad = jnp.full((E_PAD - E,), PAD_NODE, jnp.int32)
    src2d = jnp.concatenate([ei[0], pad]).reshape(E_ROWS, 128)
    dst2d = jnp.concatenate([ei[1], pad]).reshape(E_ROWS, 128)
    x_pad = jnp.pad(x, ((0, N_PAD - N), (0, 0)))

    partials = _deg_kernel(dst2d)
    pt = partials.T

    y1_lo, y1_hi = _tc1(x_pad, W1, pt)
    a1_lo, a1_hi = _scatter_h(y1_lo, y1_hi, src2d, dst2d)
    y2 = _tc2(a1_lo, a1_hi, y1_lo, y1_hi, pt, b1.reshape(1, D_H), W2)
    a2_0, a2_1 = _scatter_o(y2, y2, src2d, dst2d)
    out = _tc3(a2_0, a2_1, y2, pt, b2.reshape(1, D_OUT))
    return out[:N]


# speedup vs baseline: 22.0753x; 2.1898x over previous
import functools

import jax
import jax.numpy as jnp
from jax import lax
from jax.experimental import pallas as pl
from jax.experimental.pallas import tpu as pltpu
from jax.experimental.pallas import tpu_sc as plsc

N = 10000
D_IN = 128
D_H = 256
D_OUT = 128
E = 320000

N_PAD = 10240
PAD_NODE = N
E_ROWS = 2560
E_PAD = E_ROWS * 128

NC = 2
NS = 16
ROWS_W = E_ROWS // (NC * NS)
ROWS_T = E_ROWS // NS
ROWS_SC = 16
NSC = ROWS_T // ROWS_SC
ROWS_OUT = N_PAD // NS

_mesh = plsc.VectorSubcoreMesh(core_axis_name="c", subcore_axis_name="s")
_sc_params = pltpu.CompilerParams(needs_layout_passes=False)


@functools.partial(
    pl.kernel,
    out_type=jax.ShapeDtypeStruct((NC * NS, N_PAD), jnp.float32),
    mesh=_mesh,
    compiler_params=_sc_params,
    scratch_types=[
        pltpu.VMEM((ROWS_W, 128), jnp.int32),
        pltpu.VMEM((N_PAD,), jnp.float32),
    ],
)
def _deg_kernel(dst_hbm, out_hbm, idx_v, hist_v):
    c = lax.axis_index("c")
    s = lax.axis_index("s")
    w = c * NS + s

    zero16 = jnp.zeros((16,), jnp.float32)

    def zbody(i, carry):
        hist_v[pl.ds(i * 16, 16)] = zero16
        return carry

    lax.fori_loop(0, N_PAD // 16, zbody, 0)

    pltpu.sync_copy(dst_hbm.at[pl.ds(w * ROWS_W, ROWS_W)], idx_v)

    ones16 = jnp.ones((16,), jnp.float32)

    def ebody(i, carry):
        r = i // 8
        j = i % 8
        iv = idx_v[r, pl.ds(j * 16, 16)]
        plsc.addupdate_scatter(hist_v, [iv], ones16)
        return carry

    lax.fori_loop(0, ROWS_W * 8, ebody, 0)

    pltpu.sync_copy(hist_v, out_hbm.at[w])


def _make_scatter(edge_split):
    D = 128
    rows_t = E_ROWS // (NC * NS) if edge_split else E_ROWS // NS
    nsc = rows_t // ROWS_SC

    @functools.partial(
        pl.kernel,
        out_type=(
            jax.ShapeDtypeStruct((N_PAD, D), jnp.float32),
            jax.ShapeDtypeStruct((N_PAD, D), jnp.float32),
        ),
        mesh=_mesh,
        compiler_params=_sc_params,
        scratch_types=[
            pltpu.VMEM((ROWS_SC, 128), jnp.int32),
            pltpu.VMEM((ROWS_SC, 128), jnp.int32),
            pltpu.VMEM((2, 128, D), jnp.float32),
            pltpu.VMEM_SHARED((N_PAD, D), jnp.float32),
            pltpu.SemaphoreType.DMA,
            pltpu.SemaphoreType.DMA,
            pltpu.SemaphoreType.DMA,
            pltpu.SemaphoreType.DMA,
        ],
    )
    def scat(ylo, yhi, src_hbm, dst_hbm, olo, ohi,
             src_v, dst_v, bufs, acc_sh, g0, g1, s0, s1):
        c = lax.axis_index("c")
        s = lax.axis_index("s")
        gsems = (g0, g1)
        ssems = (s0, s1)

        rb = (c * NS + s) * rows_t if edge_split else s * rows_t

        zero16 = jnp.zeros((16,), jnp.float32)

        def zbody(i, carry):
            r = i // (D // 16)
            j = i % (D // 16)
            bufs[0, r, pl.ds(j * 16, 16)] = zero16
            return carry

        lax.fori_loop(0, 128 * (D // 16), zbody, 0)
        ob = s * ROWS_OUT
        for k in range(ROWS_OUT // 128):
            pltpu.sync_copy(bufs.at[0], acc_sh.at[pl.ds(ob + k * 128, 128)])
        plsc.subcore_barrier()

        def body(ytab, otab):
            def fire_gather(b, j):
                pltpu.async_copy(ytab.at[src_v.at[b]], bufs.at[j], gsems[j])

            def wait_gather(b, j):
                pltpu.make_async_copy(
                    ytab.at[src_v.at[b]], bufs.at[j], gsems[j]).wait()

            def fire_scatter(b, j):
                pltpu.async_copy(
                    bufs.at[j], acc_sh.at[dst_v.at[b]], ssems[j], add=True)

            def wait_scatter(b, j):
                pltpu.make_async_copy(
                    bufs.at[j], acc_sh.at[dst_v.at[b]], ssems[j]).wait()

            def chunk_body(ci, carry):
                rbase = rb + ci * ROWS_SC
                pltpu.sync_copy(src_hbm.at[pl.ds(rbase, ROWS_SC)], src_v)
                pltpu.sync_copy(dst_hbm.at[pl.ds(rbase, ROWS_SC)], dst_v)
                fire_gather(0, 0)
                for b in range(ROWS_SC):
                    j = b % 2
                    wait_gather(b, j)
                    fire_scatter(b, j)
                    if b > 0:
                        wait_scatter(b - 1, 1 - j)
                    if b + 1 < ROWS_SC:
                        fire_gather(b + 1, 1 - j)
                wait_scatter(ROWS_SC - 1, (ROWS_SC - 1) % 2)
                return carry

            lax.fori_loop(0, nsc, chunk_body, 0)

            plsc.subcore_barrier()
            for k in range(ROWS_OUT // 128):
                pltpu.sync_copy(acc_sh.at[pl.ds(ob + k * 128, 128)],
                                otab.at[pl.ds(ob + k * 128, 128)])

        @pl.when(c == 0)
        def _():
            body(ylo, olo)

        @pl.when(c == 1)
        def _():
            body(ylo if edge_split else yhi, ohi)

    return scat


_scatter_h = _make_scatter(edge_split=False)
_scatter_o = _make_scatter(edge_split=True)


_BLK = 512
_GRID = N_PAD // _BLK


def _dis_block(pt):
    deg = jnp.sum(pt, axis=1, keepdims=True) + 1.0
    return lax.rsqrt(deg)


def _tc1_body(x_ref, w_ref, pt_ref, ylo_ref, yhi_ref):
    dis = _dis_block(pt_ref[...])
    y = jnp.dot(x_ref[...], w_ref[...],
                preferred_element_type=jnp.float32) * dis
    ylo_ref[...] = y[:, :D_H // 2]
    yhi_ref[...] = y[:, D_H // 2:]


def _tc2_body(alo_ref, ahi_ref, ylo_ref, yhi_ref, pt_ref, b1_ref, w2_ref,
              y2_ref):
    dis = _dis_block(pt_ref[...])
    pre = jnp.concatenate(
        [alo_ref[...] + ylo_ref[...], ahi_ref[...] + yhi_ref[...]], axis=1)
    h = jnp.maximum(pre * dis + b1_ref[...], 0.0)
    y2_ref[...] = jnp.dot(h, w2_ref[...],
                          preferred_element_type=jnp.float32) * dis


def _tc3_body(a0_ref, a1_ref, y2_ref, pt_ref, b2_ref, out_ref):
    dis = _dis_block(pt_ref[...])
    pre = a0_ref[...] + a1_ref[...] + y2_ref[...]
    out_ref[...] = jnp.maximum(pre * dis + b2_ref[...], 0.0)


def _row_spec(d):
    return pl.BlockSpec((_BLK, d), lambda i: (i, 0))


def _full_spec(r, d):
    return pl.BlockSpec((r, d), lambda i: (0, 0))


_tc1 = pl.pallas_call(
    _tc1_body,
    grid=(_GRID,),
    in_specs=[_row_spec(D_IN), _full_spec(D_IN, D_H), _row_spec(NC * NS)],
    out_specs=(_row_spec(D_H // 2), _row_spec(D_H // 2)),
    out_shape=(jax.ShapeDtypeStruct((N_PAD, D_H // 2), jnp.float32),
               jax.ShapeDtypeStruct((N_PAD, D_H // 2), jnp.float32)),
)

_tc2 = pl.pallas_call(
    _tc2_body,
    grid=(_GRID,),
    in_specs=[_row_spec(D_H // 2), _row_spec(D_H // 2),
              _row_spec(D_H // 2), _row_spec(D_H // 2),
              _row_spec(NC * NS), _full_spec(1, D_H), _full_spec(D_H, D_OUT)],
    out_specs=_row_spec(D_OUT),
    out_shape=jax.ShapeDtypeStruct((N_PAD, D_OUT), jnp.float32),
)

_tc3 = pl.pallas_call(
    _tc3_body,
    grid=(_GRID,),
    in_specs=[_row_spec(D_OUT), _row_spec(D_OUT), _row_spec(D_OUT),
              _row_spec(NC * NS), _full_spec(1, D_OUT)],
    out_specs=_row_spec(D_OUT),
    out_shape=jax.ShapeDtypeStruct((N_PAD, D_OUT), jnp.float32),
)


def kernel(x, edge_index, W1, b1, W2, b2):
    ei = edge_index.astype(jnp.int32)
    pad = PAD_NODE + (jnp.arange(E_PAD - E, dtype=jnp.int32) % (N_PAD - N))
    src2d = jnp.concatenate([ei[0], pad]).reshape(E_ROWS, 128)
    dst2d = jnp.concatenate([ei[1], pad]).reshape(E_ROWS, 128)
    x_pad = jnp.pad(x, ((0, N_PAD - N), (0, 0)))

    partials = _deg_kernel(dst2d)
    pt = partials.T

    y1_lo, y1_hi = _tc1(x_pad, W1, pt)
    a1_lo, a1_hi = _scatter_h(y1_lo, y1_hi, src2d, dst2d)
    y2 = _tc2(a1_lo, a1_hi, y1_lo, y1_hi, pt, b1.reshape(1, D_H), W2)
    a2_0, a2_1 = _scatter_o(y2, y2, src2d, dst2d)
    out = _tc3(a2_0, a2_1, y2, pt, b2.reshape(1, D_OUT))
    return out[:N]


# speedup vs baseline: 29.2229x; 1.3238x over previous
import functools

import jax
import jax.numpy as jnp
from jax import lax
from jax.experimental import pallas as pl
from jax.experimental.pallas import tpu as pltpu
from jax.experimental.pallas import tpu_sc as plsc

N = 10000
D_IN = 128
D_H = 256
D_OUT = 128
E = 320000

N_PAD = 10240
PAD_NODE = N
E_ROWS = 2560
E_PAD = E_ROWS * 128

NC = 2
NS = 16
ROWS_W = E_ROWS // (NC * NS)
ROWS_T = E_ROWS // NS
ROWS_SC = 16
NSC = ROWS_T // ROWS_SC
ROWS_OUT = N_PAD // NS

_mesh = plsc.VectorSubcoreMesh(core_axis_name="c", subcore_axis_name="s")
_sc_params = pltpu.CompilerParams(needs_layout_passes=False)


@functools.partial(
    pl.kernel,
    out_type=jax.ShapeDtypeStruct((NC * NS, N_PAD), jnp.float32),
    mesh=_mesh,
    compiler_params=_sc_params,
    scratch_types=[
        pltpu.VMEM((ROWS_W, 128), jnp.int32),
        pltpu.VMEM((N_PAD,), jnp.float32),
    ],
)
def _deg_kernel(dst_hbm, out_hbm, idx_v, hist_v):
    c = lax.axis_index("c")
    s = lax.axis_index("s")
    w = c * NS + s

    zero16 = jnp.zeros((16,), jnp.float32)

    def zbody(i, carry):
        hist_v[pl.ds(i * 16, 16)] = zero16
        return carry

    lax.fori_loop(0, N_PAD // 16, zbody, 0)

    pltpu.sync_copy(dst_hbm.at[pl.ds(w * ROWS_W, ROWS_W)], idx_v)

    ones16 = jnp.ones((16,), jnp.float32)

    def ebody(i, carry):
        r = i // 8
        j = i % 8
        iv = idx_v[r, pl.ds(j * 16, 16)]
        plsc.addupdate_scatter(hist_v, [iv], ones16)
        return carry

    lax.fori_loop(0, ROWS_W * 8, ebody, 0)

    pltpu.sync_copy(hist_v, out_hbm.at[w])


def _make_scatter():
    D = 128
    rows_t = E_ROWS // (NC * NS)
    nsc = rows_t // ROWS_SC

    @functools.partial(
        pl.kernel,
        out_type=(
            jax.ShapeDtypeStruct((N_PAD, D), jnp.float32),
            jax.ShapeDtypeStruct((N_PAD, D), jnp.float32),
        ),
        mesh=_mesh,
        compiler_params=_sc_params,
        scratch_types=[
            pltpu.VMEM((ROWS_SC, 128), jnp.int32),
            pltpu.VMEM((ROWS_SC, 128), jnp.int32),
            pltpu.VMEM((2, 128, D), jnp.float32),
            pltpu.VMEM_SHARED((N_PAD, D), jnp.float32),
            pltpu.SemaphoreType.DMA,
            pltpu.SemaphoreType.DMA,
            pltpu.SemaphoreType.DMA,
            pltpu.SemaphoreType.DMA,
        ],
    )
    def scat(ytab, src_hbm, dst_hbm, o0, o1,
             src_v, dst_v, bufs, acc_sh, g0, g1, s0, s1):
        c = lax.axis_index("c")
        s = lax.axis_index("s")
        gsems = (g0, g1)
        ssems = (s0, s1)

        rb = (c * NS + s) * rows_t

        zero16 = jnp.zeros((16,), jnp.float32)

        def zbody(i, carry):
            r = i // (D // 16)
            j = i % (D // 16)
            bufs[0, r, pl.ds(j * 16, 16)] = zero16
            return carry

        lax.fori_loop(0, 128 * (D // 16), zbody, 0)
        ob = s * ROWS_OUT
        for k in range(ROWS_OUT // 128):
            pltpu.sync_copy(bufs.at[0], acc_sh.at[pl.ds(ob + k * 128, 128)])
        plsc.subcore_barrier()

        def body(ytab, otab):
            def fire_gather(b, j):
                pltpu.async_copy(ytab.at[src_v.at[b]], bufs.at[j], gsems[j])

            def wait_gather(b, j):
                pltpu.make_async_copy(
                    ytab.at[src_v.at[b]], bufs.at[j], gsems[j]).wait()

            def fire_scatter(b, j):
                pltpu.async_copy(
                    bufs.at[j], acc_sh.at[dst_v.at[b]], ssems[j], add=True)

            def wait_scatter(b, j):
                pltpu.make_async_copy(
                    bufs.at[j], acc_sh.at[dst_v.at[b]], ssems[j]).wait()

            def chunk_body(ci, carry):
                rbase = rb + ci * ROWS_SC
                pltpu.sync_copy(src_hbm.at[pl.ds(rbase, ROWS_SC)], src_v)
                pltpu.sync_copy(dst_hbm.at[pl.ds(rbase, ROWS_SC)], dst_v)
                fire_gather(0, 0)
                for b in range(ROWS_SC):
                    j = b % 2
                    wait_gather(b, j)
                    fire_scatter(b, j)
                    if b > 0:
                        wait_scatter(b - 1, 1 - j)
                    if b + 1 < ROWS_SC:
                        fire_gather(b + 1, 1 - j)
                wait_scatter(ROWS_SC - 1, (ROWS_SC - 1) % 2)
                return carry

            lax.fori_loop(0, nsc, chunk_body, 0)

            plsc.subcore_barrier()
            for k in range(ROWS_OUT // 128):
                pltpu.sync_copy(acc_sh.at[pl.ds(ob + k * 128, 128)],
                                otab.at[pl.ds(ob + k * 128, 128)])

        @pl.when(c == 0)
        def _():
            body(ytab, o0)

        @pl.when(c == 1)
        def _():
            body(ytab, o1)

    return scat


_scatter = _make_scatter()


_BLK = 512
_GRID = N_PAD // _BLK


def _dis_block(pt):
    deg = jnp.sum(pt, axis=1, keepdims=True) + 1.0
    return lax.rsqrt(deg)


def _tca_body(x_ref, pt_ref, xs_ref):
    xs_ref[...] = x_ref[...] * _dis_block(pt_ref[...])


def _tcb_body(a0_ref, a1_ref, xs_ref, pt_ref, w1_ref, b1_ref, w2_ref,
              y2_ref):
    dis = _dis_block(pt_ref[...])
    mx = (a0_ref[...] + a1_ref[...] + xs_ref[...]) * dis
    h = jnp.maximum(
        jnp.dot(mx, w1_ref[...], preferred_element_type=jnp.float32)
        + b1_ref[...], 0.0)
    y2_ref[...] = jnp.dot(h, w2_ref[...],
                          preferred_element_type=jnp.float32) * dis


def _tcc_body(a0_ref, a1_ref, y2_ref, pt_ref, b2_ref, out_ref):
    dis = _dis_block(pt_ref[...])
    pre = a0_ref[...] + a1_ref[...] + y2_ref[...]
    out_ref[...] = jnp.maximum(pre * dis + b2_ref[...], 0.0)


def _row_spec(d):
    return pl.BlockSpec((_BLK, d), lambda i: (i, 0))


def _full_spec(r, d):
    return pl.BlockSpec((r, d), lambda i: (0, 0))


_tca = pl.pallas_call(
    _tca_body,
    grid=(_GRID,),
    in_specs=[_row_spec(D_IN), _row_spec(NC * NS)],
    out_specs=_row_spec(D_IN),
    out_shape=jax.ShapeDtypeStruct((N_PAD, D_IN), jnp.float32),
)

_tcb = pl.pallas_call(
    _tcb_body,
    grid=(_GRID,),
    in_specs=[_row_spec(D_IN), _row_spec(D_IN), _row_spec(D_IN),
              _row_spec(NC * NS), _full_spec(D_IN, D_H), _full_spec(1, D_H),
              _full_spec(D_H, D_OUT)],
    out_specs=_row_spec(D_OUT),
    out_shape=jax.ShapeDtypeStruct((N_PAD, D_OUT), jnp.float32),
)

_tcc = pl.pallas_call(
    _tcc_body,
    grid=(_GRID,),
    in_specs=[_row_spec(D_OUT), _row_spec(D_OUT), _row_spec(D_OUT),
              _row_spec(NC * NS), _full_spec(1, D_OUT)],
    out_specs=_row_spec(D_OUT),
    out_shape=jax.ShapeDtypeStruct((N_PAD, D_OUT), jnp.float32),
)


def kernel(x, edge_index, W1, b1, W2, b2):
    ei = edge_index.astype(jnp.int32)
    pad = PAD_NODE + (jnp.arange(E_PAD - E, dtype=jnp.int32) % (N_PAD - N))
    src2d = jnp.concatenate([ei[0], pad]).reshape(E_ROWS, 128)
    dst2d = jnp.concatenate([ei[1], pad]).reshape(E_ROWS, 128)
    x_pad = jnp.pad(x, ((0, N_PAD - N), (0, 0)))

    partials = _deg_kernel(dst2d)
    pt = partials.T

    xs = _tca(x_pad, pt)
    a1_0, a1_1 = _scatter(xs, src2d, dst2d)
    y2 = _tcb(a1_0, a1_1, xs, pt, W1, b1.reshape(1, D_H), W2)
    a2_0, a2_1 = _scatter(y2, src2d, dst2d)
    out = _tcc(a2_0, a2_1, y2, pt, b2.reshape(1, D_OUT))
    return out[:N]


# speedup vs baseline: 30.2607x; 1.0355x over previous
import functools

import jax
import jax.numpy as jnp
from jax import lax
from jax.experimental import pallas as pl
from jax.experimental.pallas import tpu as pltpu
from jax.experimental.pallas import tpu_sc as plsc

N = 10000
D_IN = 128
D_H = 256
D_OUT = 128
E = 320000

N_PAD = 10240
PAD_NODE = N
E_ROWS = 2560
E_PAD = E_ROWS * 128

NC = 2
NS = 16
ROWS_W = E_ROWS // (NC * NS)
ROWS_T = E_ROWS // NS
ROWS_SC = 40
NSC = ROWS_T // ROWS_SC
ROWS_OUT = N_PAD // NS

_mesh = plsc.VectorSubcoreMesh(core_axis_name="c", subcore_axis_name="s")
_sc_params = pltpu.CompilerParams(needs_layout_passes=False)


@functools.partial(
    pl.kernel,
    out_type=jax.ShapeDtypeStruct((NC * NS, N_PAD), jnp.float32),
    mesh=_mesh,
    compiler_params=_sc_params,
    scratch_types=[
        pltpu.VMEM((ROWS_W, 128), jnp.int32),
        pltpu.VMEM((N_PAD,), jnp.float32),
    ],
)
def _deg_kernel(dst_hbm, out_hbm, idx_v, hist_v):
    c = lax.axis_index("c")
    s = lax.axis_index("s")
    w = c * NS + s

    zero16 = jnp.zeros((16,), jnp.float32)

    def zbody(i, carry):
        hist_v[pl.ds(i * 16, 16)] = zero16
        return carry

    lax.fori_loop(0, N_PAD // 16, zbody, 0)

    pltpu.sync_copy(dst_hbm.at[pl.ds(w * ROWS_W, ROWS_W)], idx_v)

    ones16 = jnp.ones((16,), jnp.float32)

    def ebody(i, carry):
        r = i // 8
        j = i % 8
        iv = idx_v[r, pl.ds(j * 16, 16)]
        plsc.addupdate_scatter(hist_v, [iv], ones16)
        return carry

    lax.fori_loop(0, ROWS_W * 8, ebody, 0)

    pltpu.sync_copy(hist_v, out_hbm.at[w])


def _make_scatter():
    D = 128
    rows_t = E_ROWS // (NC * NS)
    nsc = rows_t // ROWS_SC

    @functools.partial(
        pl.kernel,
        out_type=(
            jax.ShapeDtypeStruct((N_PAD, D), jnp.float32),
            jax.ShapeDtypeStruct((N_PAD, D), jnp.float32),
        ),
        mesh=_mesh,
        compiler_params=_sc_params,
        scratch_types=[
            pltpu.VMEM((ROWS_SC, 128), jnp.int32),
            pltpu.VMEM((ROWS_SC, 128), jnp.int32),
            pltpu.VMEM((2, 128, D), jnp.float32),
            pltpu.VMEM_SHARED((N_PAD, D), jnp.float32),
            pltpu.SemaphoreType.DMA,
            pltpu.SemaphoreType.DMA,
            pltpu.SemaphoreType.DMA,
            pltpu.SemaphoreType.DMA,
        ],
    )
    def scat(ytab, src_hbm, dst_hbm, o0, o1,
             src_v, dst_v, bufs, acc_sh, g0, g1, s0, s1):
        c = lax.axis_index("c")
        s = lax.axis_index("s")
        gsems = (g0, g1)
        ssems = (s0, s1)

        rb = (c * NS + s) * rows_t

        zero16 = jnp.zeros((16,), jnp.float32)

        def zbody(i, carry):
            r = i // (D // 16)
            j = i % (D // 16)
            bufs[0, r, pl.ds(j * 16, 16)] = zero16
            return carry

        lax.fori_loop(0, 128 * (D // 16), zbody, 0)
        ob = s * ROWS_OUT
        for k in range(ROWS_OUT // 128):
            pltpu.sync_copy(bufs.at[0], acc_sh.at[pl.ds(ob + k * 128, 128)])
        plsc.subcore_barrier()

        def body(ytab, otab):
            def fire_gather(b, j):
                pltpu.async_copy(ytab.at[src_v.at[b]], bufs.at[j], gsems[j])

            def wait_gather(b, j):
                pltpu.make_async_copy(
                    ytab.at[src_v.at[b]], bufs.at[j], gsems[j]).wait()

            def fire_scatter(b, j):
                pltpu.async_copy(
                    bufs.at[j], acc_sh.at[dst_v.at[b]], ssems[j], add=True)

            def wait_scatter(b, j):
                pltpu.make_async_copy(
                    bufs.at[j], acc_sh.at[dst_v.at[b]], ssems[j]).wait()

            def chunk_body(ci, carry):
                rbase = rb + ci * ROWS_SC
                cp1 = pltpu.async_copy(
                    src_hbm.at[pl.ds(rbase, ROWS_SC)], src_v, s0)
                cp2 = pltpu.async_copy(
                    dst_hbm.at[pl.ds(rbase, ROWS_SC)], dst_v, s1)
                cp1.wait()
                cp2.wait()
                fire_gather(0, 0)
                for b in range(ROWS_SC):
                    j = b % 2
                    wait_gather(b, j)
                    fire_scatter(b, j)
                    if b > 0:
                        wait_scatter(b - 1, 1 - j)
                    if b + 1 < ROWS_SC:
                        fire_gather(b + 1, 1 - j)
                wait_scatter(ROWS_SC - 1, (ROWS_SC - 1) % 2)
                return carry

            lax.fori_loop(0, nsc, chunk_body, 0)

            plsc.subcore_barrier()
            for k in range(ROWS_OUT // 128):
                pltpu.sync_copy(acc_sh.at[pl.ds(ob + k * 128, 128)],
                                otab.at[pl.ds(ob + k * 128, 128)])

        @pl.when(c == 0)
        def _():
            body(ytab, o0)

        @pl.when(c == 1)
        def _():
            body(ytab, o1)

    return scat


_scatter = _make_scatter()


_BLK = 512
_GRID = N_PAD // _BLK


def _dis_block(pt):
    deg = jnp.sum(pt, axis=1, keepdims=True) + 1.0
    return lax.rsqrt(deg)


def _tca_body(x_ref, pt_ref, xs_ref):
    xs_ref[...] = x_ref[...] * _dis_block(pt_ref[...])


def _tcb_body(a0_ref, a1_ref, xs_ref, pt_ref, w1_ref, b1_ref, w2_ref,
              y2_ref):
    dis = _dis_block(pt_ref[...])
    mx = (a0_ref[...] + a1_ref[...] + xs_ref[...]) * dis
    h = jnp.maximum(
        jnp.dot(mx, w1_ref[...], preferred_element_type=jnp.float32)
        + b1_ref[...], 0.0)
    y2_ref[...] = jnp.dot(h, w2_ref[...],
                          preferred_element_type=jnp.float32) * dis


def _tcc_body(a0_ref, a1_ref, y2_ref, pt_ref, b2_ref, out_ref):
    dis = _dis_block(pt_ref[...])
    pre = a0_ref[...] + a1_ref[...] + y2_ref[...]
    out_ref[...] = jnp.maximum(pre * dis + b2_ref[...], 0.0)


def _row_spec(d):
    return pl.BlockSpec((_BLK, d), lambda i: (i, 0))


def _full_spec(r, d):
    return pl.BlockSpec((r, d), lambda i: (0, 0))


_tca = pl.pallas_call(
    _tca_body,
    grid=(_GRID,),
    in_specs=[_row_spec(D_IN), _row_spec(NC * NS)],
    out_specs=_row_spec(D_IN),
    out_shape=jax.ShapeDtypeStruct((N_PAD, D_IN), jnp.float32),
)

_tcb = pl.pallas_call(
    _tcb_body,
    grid=(_GRID,),
    in_specs=[_row_spec(D_IN), _row_spec(D_IN), _row_spec(D_IN),
              _row_spec(NC * NS), _full_spec(D_IN, D_H), _full_spec(1, D_H),
              _full_spec(D_H, D_OUT)],
    out_specs=_row_spec(D_OUT),
    out_shape=jax.ShapeDtypeStruct((N_PAD, D_OUT), jnp.float32),
)

_tcc = pl.pallas_call(
    _tcc_body,
    grid=(_GRID,),
    in_specs=[_row_spec(D_OUT), _row_spec(D_OUT), _row_spec(D_OUT),
              _row_spec(NC * NS), _full_spec(1, D_OUT)],
    out_specs=_row_spec(D_OUT),
    out_shape=jax.ShapeDtypeStruct((N_PAD, D_OUT), jnp.float32),
)


def kernel(x, edge_index, W1, b1, W2, b2):
    ei = edge_index.astype(jnp.int32)
    pad = PAD_NODE + (jnp.arange(E_PAD - E, dtype=jnp.int32) % (N_PAD - N))
    src2d = jnp.concatenate([ei[0], pad]).reshape(E_ROWS, 128)
    dst2d = jnp.concatenate([ei[1], pad]).reshape(E_ROWS, 128)
    x_pad = jnp.pad(x, ((0, N_PAD - N), (0, 0)))

    partials = _deg_kernel(dst2d)
    pt = partials.T

    xs = _tca(x_pad, pt)
    a1_0, a1_1 = _scatter(xs, src2d, dst2d)
    y2 = _tcb(a1_0, a1_1, xs, pt, W1, b1.reshape(1, D_H), W2)
    a2_0, a2_1 = _scatter(y2, src2d, dst2d)
    out = _tcc(a2_0, a2_1, y2, pt, b2.reshape(1, D_OUT))
    return out[:N]


# speedup vs baseline: 32.1664x; 1.0630x over previous
import functools

import jax
import jax.numpy as jnp
from jax import lax
from jax.experimental import pallas as pl
from jax.experimental.pallas import tpu as pltpu
from jax.experimental.pallas import tpu_sc as plsc

N = 10000
D_IN = 128
D_H = 256
D_OUT = 128
E = 320000

N_PAD = 10240
PAD_NODE = N
E_ROWS = 2560
E_PAD = E_ROWS * 128

NC = 2
NS = 16
ROWS_W = E_ROWS // (NC * NS)
ROWS_T = E_ROWS // NS
ROWS_SC = 40
NSC = ROWS_T // ROWS_SC
ROWS_OUT = N_PAD // NS

_mesh = plsc.VectorSubcoreMesh(core_axis_name="c", subcore_axis_name="s")
_sc_params = pltpu.CompilerParams(needs_layout_passes=False)


@functools.partial(
    pl.kernel,
    out_type=jax.ShapeDtypeStruct((NC * NS, N_PAD), jnp.float32),
    mesh=_mesh,
    compiler_params=_sc_params,
    scratch_types=[
        pltpu.VMEM((ROWS_W, 128), jnp.int32),
        pltpu.VMEM((N_PAD,), jnp.float32),
    ],
)
def _deg_kernel(dst_hbm, out_hbm, idx_v, hist_v):
    c = lax.axis_index("c")
    s = lax.axis_index("s")
    w = c * NS + s

    zero16 = jnp.zeros((16,), jnp.float32)

    def zbody(i, carry):
        hist_v[pl.ds(i * 16, 16)] = zero16
        return carry

    lax.fori_loop(0, N_PAD // 16, zbody, 0)

    pltpu.sync_copy(dst_hbm.at[pl.ds(w * ROWS_W, ROWS_W)], idx_v)

    ones16 = jnp.ones((16,), jnp.float32)

    def ebody(i, carry):
        r = i // 8
        j = i % 8
        iv = idx_v[r, pl.ds(j * 16, 16)]
        plsc.addupdate_scatter(hist_v, [iv], ones16)
        return carry

    lax.fori_loop(0, ROWS_W * 8, ebody, 0)

    pltpu.sync_copy(hist_v, out_hbm.at[w])


def _make_scatter():
    D = 128
    rows_t = E_ROWS // (NC * NS)
    nsc = rows_t // ROWS_SC

    @functools.partial(
        pl.kernel,
        out_type=(
            jax.ShapeDtypeStruct((N_PAD, D), jnp.float32),
            jax.ShapeDtypeStruct((N_PAD, D), jnp.float32),
        ),
        mesh=_mesh,
        compiler_params=_sc_params,
        scratch_types=[
            pltpu.VMEM((ROWS_SC, 128), jnp.int32),
            pltpu.VMEM((ROWS_SC, 128), jnp.int32),
            pltpu.VMEM((4, 64, D), jnp.float32),
            pltpu.VMEM_SHARED((N_PAD, D), jnp.float32),
            pltpu.SemaphoreType.DMA,
            pltpu.SemaphoreType.DMA,
            pltpu.SemaphoreType.DMA,
            pltpu.SemaphoreType.DMA,
            pltpu.SemaphoreType.DMA,
            pltpu.SemaphoreType.DMA,
            pltpu.SemaphoreType.DMA,
            pltpu.SemaphoreType.DMA,
        ],
    )
    def scat(ytab, src_hbm, dst_hbm, o0, o1,
             src_v, dst_v, bufs, acc_sh, g0, g1, g2, g3, s0, s1, s2, s3):
        c = lax.axis_index("c")
        s = lax.axis_index("s")
        gsems = (g0, g1, g2, g3)
        ssems = (s0, s1, s2, s3)

        rb = (c * NS + s) * rows_t

        zero16 = jnp.zeros((16,), jnp.float32)

        def zbody(i, carry):
            r = i // (D // 16)
            j = i % (D // 16)
            bufs[0, r, pl.ds(j * 16, 16)] = zero16
            return carry

        lax.fori_loop(0, 64 * (D // 16), zbody, 0)
        ob = s * ROWS_OUT
        for k in range(ROWS_OUT // 64):
            pltpu.sync_copy(bufs.at[0], acc_sh.at[pl.ds(ob + k * 64, 64)])
        plsc.subcore_barrier()

        def body(ytab, otab):
            def gidx(r, h):
                return src_v.at[r, pl.ds(h * 64, 64)]

            def didx(r, h):
                return dst_v.at[r, pl.ds(h * 64, 64)]

            def fire_gather(r, h, j):
                pltpu.async_copy(ytab.at[gidx(r, h)], bufs.at[j], gsems[j])

            def wait_gather(r, h, j):
                pltpu.make_async_copy(
                    ytab.at[gidx(r, h)], bufs.at[j], gsems[j]).wait()

            def fire_scatter(r, h, j):
                pltpu.async_copy(
                    bufs.at[j], acc_sh.at[didx(r, h)], ssems[j], add=True)

            def wait_scatter(r, h, j):
                pltpu.make_async_copy(
                    bufs.at[j], acc_sh.at[didx(r, h)], ssems[j]).wait()

            nq = ROWS_SC // 2

            def quad(q, carry):
                for m in range(4):
                    r = 2 * q + m // 2
                    h = m % 2
                    wait_gather(r, h, m)
                    fire_scatter(r, h, m)
                    jn = (m + 2) % 4
                    r2 = 2 * q + (m - 2) // 2
                    h2 = (m - 2) % 2
                    rn = 2 * q + (m + 2) // 2
                    if m < 2:
                        @pl.when(q > 0)
                        def _():
                            wait_scatter(r2, h2, jn)
                        fire_gather(rn, h, jn)
                    else:
                        wait_scatter(r2, h2, jn)

                        @pl.when(q < nq - 1)
                        def _():
                            fire_gather(rn, h, jn)
                return carry

            def chunk_body(ci, carry):
                rbase = rb + ci * ROWS_SC
                cp1 = pltpu.async_copy(
                    src_hbm.at[pl.ds(rbase, ROWS_SC)], src_v, g0)
                cp2 = pltpu.async_copy(
                    dst_hbm.at[pl.ds(rbase, ROWS_SC)], dst_v, g1)
                cp1.wait()
                cp2.wait()
                fire_gather(0, 0, 0)
                fire_gather(0, 1, 1)
                lax.fori_loop(0, nq, quad, 0)
                wait_scatter(ROWS_SC - 1, 0, 2)
                wait_scatter(ROWS_SC - 1, 1, 3)
                return carry

            lax.fori_loop(0, nsc, chunk_body, 0)

            plsc.subcore_barrier()
            for k in range(ROWS_OUT // 128):
                pltpu.sync_copy(acc_sh.at[pl.ds(ob + k * 128, 128)],
                                otab.at[pl.ds(ob + k * 128, 128)])

        @pl.when(c == 0)
        def _():
            body(ytab, o0)

        @pl.when(c == 1)
        def _():
            body(ytab, o1)

    return scat


_scatter = _make_scatter()


_BLK = 512
_GRID = N_PAD // _BLK


def _dis_block(pt):
    deg = jnp.sum(pt, axis=1, keepdims=True) + 1.0
    return lax.rsqrt(deg)


def _tca_body(x_ref, pt_ref, xs_ref):
    xs_ref[...] = x_ref[...] * _dis_block(pt_ref[...])


def _tcb_body(a0_ref, a1_ref, xs_ref, pt_ref, w1_ref, b1_ref, w2_ref,
              y2_ref):
    dis = _dis_block(pt_ref[...])
    mx = (a0_ref[...] + a1_ref[...] + xs_ref[...]) * dis
    h = jnp.maximum(
        jnp.dot(mx, w1_ref[...], preferred_element_type=jnp.float32)
        + b1_ref[...], 0.0)
    y2_ref[...] = jnp.dot(h, w2_ref[...],
                          preferred_element_type=jnp.float32) * dis


def _tcc_body(a0_ref, a1_ref, y2_ref, pt_ref, b2_ref, out_ref):
    dis = _dis_block(pt_ref[...])
    pre = a0_ref[...] + a1_ref[...] + y2_ref[...]
    out_ref[...] = jnp.maximum(pre * dis + b2_ref[...], 0.0)


def _row_spec(d):
    return pl.BlockSpec((_BLK, d), lambda i: (i, 0))


def _full_spec(r, d):
    return pl.BlockSpec((r, d), lambda i: (0, 0))


_tca = pl.pallas_call(
    _tca_body,
    grid=(_GRID,),
    in_specs=[_row_spec(D_IN), _row_spec(NC * NS)],
    out_specs=_row_spec(D_IN),
    out_shape=jax.ShapeDtypeStruct((N_PAD, D_IN), jnp.float32),
)

_tcb = pl.pallas_call(
    _tcb_body,
    grid=(_GRID,),
    in_specs=[_row_spec(D_IN), _row_spec(D_IN), _row_spec(D_IN),
              _row_spec(NC * NS), _full_spec(D_IN, D_H), _full_spec(1, D_H),
              _full_spec(D_H, D_OUT)],
    out_specs=_row_spec(D_OUT),
    out_shape=jax.ShapeDtypeStruct((N_PAD, D_OUT), jnp.float32),
)

_tcc = pl.pallas_call(
    _tcc_body,
    grid=(_GRID,),
    in_specs=[_row_spec(D_OUT), _row_spec(D_OUT), _row_spec(D_OUT),
              _row_spec(NC * NS), _full_spec(1, D_OUT)],
    out_specs=_row_spec(D_OUT),
    out_shape=jax.ShapeDtypeStruct((N_PAD, D_OUT), jnp.float32),
)


def kernel(x, edge_index, W1, b1, W2, b2):
    ei = edge_index.astype(jnp.int32)
    pad = PAD_NODE + (jnp.arange(E_PAD - E, dtype=jnp.int32) % (N_PAD - N))
    src2d = jnp.concatenate([ei[0], pad]).reshape(E_ROWS, 128)
    dst2d = jnp.concatenate([ei[1], pad]).reshape(E_ROWS, 128)
    x_pad = jnp.pad(x, ((0, N_PAD - N), (0, 0)))

    partials = _deg_kernel(dst2d)
    pt = partials.T

    xs = _tca(x_pad, pt)
    a1_0, a1_1 = _scatter(xs, src2d, dst2d)
    y2 = _tcb(a1_0, a1_1, xs, pt, W1, b1.reshape(1, D_H), W2)
    a2_0, a2_1 = _scatter(y2, src2d, dst2d)
    out = _tcc(a2_0, a2_1, y2, pt, b2.reshape(1, D_OUT))
    return out[:N]


# speedup vs baseline: 33.6373x; 1.0457x over previous
import functools

import jax
import jax.numpy as jnp
from jax import lax
from jax.experimental import pallas as pl
from jax.experimental.pallas import tpu as pltpu
from jax.experimental.pallas import tpu_sc as plsc

N = 10000
D_IN = 128
D_H = 256
D_OUT = 128
E = 320000

N_PAD = 10240
PAD_NODE = N
E_ROWS = 2560
E_PAD = E_ROWS * 128

NC = 2
NS = 16
ROWS_W = E_ROWS // (NC * NS)
ROWS_T = E_ROWS // NS
ROWS_SC = 40
BATCH = 32
NBUF = 8
NSC = ROWS_T // ROWS_SC
ROWS_OUT = N_PAD // NS

_mesh = plsc.VectorSubcoreMesh(core_axis_name="c", subcore_axis_name="s")
_sc_params = pltpu.CompilerParams(needs_layout_passes=False)


@functools.partial(
    pl.kernel,
    out_type=jax.ShapeDtypeStruct((NC * NS, N_PAD), jnp.float32),
    mesh=_mesh,
    compiler_params=_sc_params,
    scratch_types=[
        pltpu.VMEM((ROWS_W, 128), jnp.int32),
        pltpu.VMEM((N_PAD,), jnp.float32),
    ],
)
def _deg_kernel(dst_hbm, out_hbm, idx_v, hist_v):
    c = lax.axis_index("c")
    s = lax.axis_index("s")
    w = c * NS + s

    zero16 = jnp.zeros((16,), jnp.float32)

    def zbody(i, carry):
        hist_v[pl.ds(i * 16, 16)] = zero16
        return carry

    lax.fori_loop(0, N_PAD // 16, zbody, 0)

    pltpu.sync_copy(dst_hbm.at[pl.ds(w * ROWS_W, ROWS_W)], idx_v)

    ones16 = jnp.ones((16,), jnp.float32)

    def ebody(i, carry):
        r = i // 8
        j = i % 8
        iv = idx_v[r, pl.ds(j * 16, 16)]
        plsc.addupdate_scatter(hist_v, [iv], ones16)
        return carry

    lax.fori_loop(0, ROWS_W * 8, ebody, 0)

    pltpu.sync_copy(hist_v, out_hbm.at[w])


def _make_scatter():
    D = 128
    rows_t = E_ROWS // (NC * NS)
    nsc = rows_t // ROWS_SC

    @functools.partial(
        pl.kernel,
        out_type=(
            jax.ShapeDtypeStruct((N_PAD, D), jnp.float32),
            jax.ShapeDtypeStruct((N_PAD, D), jnp.float32),
        ),
        mesh=_mesh,
        compiler_params=_sc_params,
        scratch_types=[
            pltpu.VMEM((ROWS_SC, 128), jnp.int32),
            pltpu.VMEM((ROWS_SC, 128), jnp.int32),
            pltpu.VMEM((NBUF, BATCH, D), jnp.float32),
            pltpu.VMEM_SHARED((N_PAD, D), jnp.float32),
        ] + [pltpu.SemaphoreType.DMA] * (2 * NBUF),
    )
    def scat(ytab, src_hbm, dst_hbm, o0, o1,
             src_v, dst_v, bufs, acc_sh, *sems):
        c = lax.axis_index("c")
        s = lax.axis_index("s")
        gsems = sems[:NBUF]
        ssems = sems[NBUF:]

        rb = (c * NS + s) * rows_t

        zero16 = jnp.zeros((16,), jnp.float32)

        def zbody(i, carry):
            r = i // (D // 16)
            j = i % (D // 16)
            bufs[0, r, pl.ds(j * 16, 16)] = zero16
            return carry

        lax.fori_loop(0, BATCH * (D // 16), zbody, 0)
        ob = s * ROWS_OUT
        for k in range(ROWS_OUT // BATCH):
            pltpu.sync_copy(bufs.at[0],
                            acc_sh.at[pl.ds(ob + k * BATCH, BATCH)])
        plsc.subcore_barrier()

        def body(ytab, otab):
            def gidx(r, h):
                return src_v.at[r, pl.ds(h * BATCH, BATCH)]

            def didx(r, h):
                return dst_v.at[r, pl.ds(h * BATCH, BATCH)]

            def fire_gather(r, h, j):
                pltpu.async_copy(ytab.at[gidx(r, h)], bufs.at[j], gsems[j])

            def wait_gather(r, h, j):
                pltpu.make_async_copy(
                    ytab.at[gidx(r, h)], bufs.at[j], gsems[j]).wait()

            def fire_scatter(r, h, j):
                pltpu.async_copy(
                    bufs.at[j], acc_sh.at[didx(r, h)], ssems[j], add=True)

            def wait_scatter(r, h, j):
                pltpu.make_async_copy(
                    bufs.at[j], acc_sh.at[didx(r, h)], ssems[j]).wait()

            BPR = 128 // BATCH
            RPG = NBUF // BPR
            L = NBUF // 2
            nq = ROWS_SC // RPG

            def group(q, carry):
                for m in range(NBUF):
                    r = RPG * q + m // BPR
                    h = m % BPR
                    wait_gather(r, h, m)
                    fire_scatter(r, h, m)
                    jn = (m + L) % NBUF
                    r2 = RPG * q + (m - L) // BPR
                    h2 = (m - L) % BPR
                    rn = RPG * q + (m + L) // BPR
                    hn = (m + L) % BPR
                    if m < L:
                        @pl.when(q > 0)
                        def _():
                            wait_scatter(r2, h2, jn)
                        fire_gather(rn, hn, jn)
                    else:
                        wait_scatter(r2, h2, jn)

                        @pl.when(q < nq - 1)
                        def _():
                            fire_gather(rn, hn, jn)
                return carry

            def chunk_body(ci, carry):
                rbase = rb + ci * ROWS_SC
                cp1 = pltpu.async_copy(
                    src_hbm.at[pl.ds(rbase, ROWS_SC)], src_v, gsems[0])
                cp2 = pltpu.async_copy(
                    dst_hbm.at[pl.ds(rbase, ROWS_SC)], dst_v, gsems[1])
                cp1.wait()
                cp2.wait()
                for b in range(L):
                    fire_gather(b // BPR, b % BPR, b)
                lax.fori_loop(0, nq, group, 0)
                nb = ROWS_SC * BPR
                for b in range(nb - L, nb):
                    wait_scatter(b // BPR, b % BPR, b % NBUF)
                return carry

            lax.fori_loop(0, nsc, chunk_body, 0)

            plsc.subcore_barrier()
            for k in range(ROWS_OUT // 128):
                pltpu.sync_copy(acc_sh.at[pl.ds(ob + k * 128, 128)],
                                otab.at[pl.ds(ob + k * 128, 128)])

        @pl.when(c == 0)
        def _():
            body(ytab, o0)

        @pl.when(c == 1)
        def _():
            body(ytab, o1)

    return scat


_scatter = _make_scatter()


_BLK = 512
_GRID = N_PAD // _BLK


def _dis_block(pt):
    deg = jnp.sum(pt, axis=1, keepdims=True) + 1.0
    return lax.rsqrt(deg)


def _tca_body(x_ref, pt_ref, xs_ref):
    xs_ref[...] = x_ref[...] * _dis_block(pt_ref[...])


def _tcb_body(a0_ref, a1_ref, xs_ref, pt_ref, w1_ref, b1_ref, w2_ref,
              y2_ref):
    dis = _dis_block(pt_ref[...])
    mx = (a0_ref[...] + a1_ref[...] + xs_ref[...]) * dis
    h = jnp.maximum(
        jnp.dot(mx, w1_ref[...], preferred_element_type=jnp.float32)
        + b1_ref[...], 0.0)
    y2_ref[...] = jnp.dot(h, w2_ref[...],
                          preferred_element_type=jnp.float32) * dis


def _tcc_body(a0_ref, a1_ref, y2_ref, pt_ref, b2_ref, out_ref):
    dis = _dis_block(pt_ref[...])
    pre = a0_ref[...] + a1_ref[...] + y2_ref[...]
    out_ref[...] = jnp.maximum(pre * dis + b2_ref[...], 0.0)


def _row_spec(d):
    return pl.BlockSpec((_BLK, d), lambda i: (i, 0))


def _full_spec(r, d):
    return pl.BlockSpec((r, d), lambda i: (0, 0))


_tca = pl.pallas_call(
    _tca_body,
    grid=(_GRID,),
    in_specs=[_row_spec(D_IN), _row_spec(NC * NS)],
    out_specs=_row_spec(D_IN),
    out_shape=jax.ShapeDtypeStruct((N_PAD, D_IN), jnp.float32),
)

_tcb = pl.pallas_call(
    _tcb_body,
    grid=(_GRID,),
    in_specs=[_row_spec(D_IN), _row_spec(D_IN), _row_spec(D_IN),
              _row_spec(NC * NS), _full_spec(D_IN, D_H), _full_spec(1, D_H),
              _full_spec(D_H, D_OUT)],
    out_specs=_row_spec(D_OUT),
    out_shape=jax.ShapeDtypeStruct((N_PAD, D_OUT), jnp.float32),
)

_tcc = pl.pallas_call(
    _tcc_body,
    grid=(_GRID,),
    in_specs=[_row_spec(D_OUT), _row_spec(D_OUT), _row_spec(D_OUT),
              _row_spec(NC * NS), _full_spec(1, D_OUT)],
    out_specs=_row_spec(D_OUT),
    out_shape=jax.ShapeDtypeStruct((N_PAD, D_OUT), jnp.float32),
)


def kernel(x, edge_index, W1, b1, W2, b2):
    ei = edge_index.astype(jnp.int32)
    pad = PAD_NODE + (jnp.arange(E_PAD - E, dtype=jnp.int32) % (N_PAD - N))
    src2d = jnp.concatenate([ei[0], pad]).reshape(E_ROWS, 128)
    dst2d = jnp.concatenate([ei[1], pad]).reshape(E_ROWS, 128)
    x_pad = jnp.pad(x, ((0, N_PAD - N), (0, 0)))

    partials = _deg_kernel(dst2d)
    pt = partials.T

    xs = _tca(x_pad, pt)
    a1_0, a1_1 = _scatter(xs, src2d, dst2d)
    y2 = _tcb(a1_0, a1_1, xs, pt, W1, b1.reshape(1, D_H), W2)
    a2_0, a2_1 = _scatter(y2, src2d, dst2d)
    out = _tcc(a2_0, a2_1, y2, pt, b2.reshape(1, D_OUT))
    return out[:N]


# speedup vs baseline: 35.3102x; 1.0497x over previous
import functools

import jax
import jax.numpy as jnp
from jax import lax
from jax.experimental import pallas as pl
from jax.experimental.pallas import tpu as pltpu
from jax.experimental.pallas import tpu_sc as plsc

N = 10000
D_IN = 128
D_H = 256
D_OUT = 128
E = 320000

N_PAD = 10240
PAD_NODE = N
E_ROWS = 2560
E_PAD = E_ROWS * 128

NC = 2
NS = 16
ROWS_W = E_ROWS // (NC * NS)
ROWS_T = E_ROWS // NS
ROWS_SC = 40
BATCH = 32
NBUF = 8
NSC = ROWS_T // ROWS_SC
ROWS_OUT = N_PAD // NS

_mesh = plsc.VectorSubcoreMesh(core_axis_name="c", subcore_axis_name="s")
_sc_params = pltpu.CompilerParams(needs_layout_passes=False)


@functools.partial(
    pl.kernel,
    out_type=jax.ShapeDtypeStruct((NC * NS, N_PAD), jnp.float32),
    mesh=_mesh,
    compiler_params=_sc_params,
    scratch_types=[
        pltpu.VMEM((ROWS_W, 128), jnp.int32),
        pltpu.VMEM((N_PAD,), jnp.float32),
    ],
)
def _deg_kernel(dst_hbm, out_hbm, idx_v, hist_v):
    c = lax.axis_index("c")
    s = lax.axis_index("s")
    w = c * NS + s

    zero16 = jnp.zeros((16,), jnp.float32)

    def zbody(i, carry):
        hist_v[pl.ds(i * 16, 16)] = zero16
        return carry

    lax.fori_loop(0, N_PAD // 16, zbody, 0)

    pltpu.sync_copy(dst_hbm.at[pl.ds(w * ROWS_W, ROWS_W)], idx_v)

    ones16 = jnp.ones((16,), jnp.float32)

    def ebody(i, carry):
        r = i // 8
        j = i % 8
        iv = idx_v[r, pl.ds(j * 16, 16)]
        plsc.addupdate_scatter(hist_v, [iv], ones16)
        return carry

    lax.fori_loop(0, ROWS_W * 8, ebody, 0)

    pltpu.sync_copy(hist_v, out_hbm.at[w])


def _make_scatter():
    D = 128
    rows_t = E_ROWS // (NC * NS)
    nsc = rows_t // ROWS_SC

    @functools.partial(
        pl.kernel,
        out_type=(
            jax.ShapeDtypeStruct((N_PAD, D), jnp.float32),
            jax.ShapeDtypeStruct((N_PAD, D), jnp.float32),
        ),
        mesh=_mesh,
        compiler_params=_sc_params,
        scratch_types=[
            pltpu.VMEM((ROWS_SC, 128), jnp.int32),
            pltpu.VMEM((ROWS_SC, 128), jnp.int32),
            pltpu.VMEM((NBUF, BATCH, D), jnp.float32),
            pltpu.VMEM_SHARED((N_PAD, D), jnp.float32),
        ] + [pltpu.SemaphoreType.DMA] * (2 * NBUF),
    )
    def scat(ytab, src_hbm, dst_hbm, o0, o1,
             src_v, dst_v, bufs, acc_sh, *sems):
        c = lax.axis_index("c")
        s = lax.axis_index("s")
        gsems = sems[:NBUF]
        ssems = sems[NBUF:]

        rb = (c * NS + s) * rows_t

        zero16 = jnp.zeros((16,), jnp.float32)

        def zbody(i, carry):
            r = i // (D // 16)
            j = i % (D // 16)
            bufs[0, r, pl.ds(j * 16, 16)] = zero16
            return carry

        lax.fori_loop(0, BATCH * (D // 16), zbody, 0)
        ob = s * ROWS_OUT
        for k in range(ROWS_OUT // BATCH):
            pltpu.sync_copy(bufs.at[0],
                            acc_sh.at[pl.ds(ob + k * BATCH, BATCH)])
        plsc.subcore_barrier()

        def body(ytab, otab):
            def gidx(r, h):
                return src_v.at[r, pl.ds(h * BATCH, BATCH)]

            def didx(r, h):
                return dst_v.at[r, pl.ds(h * BATCH, BATCH)]

            def fire_gather(r, h, j):
                pltpu.async_copy(ytab.at[gidx(r, h)], bufs.at[j], gsems[j])

            def wait_gather(r, h, j):
                pltpu.make_async_copy(
                    ytab.at[gidx(r, h)], bufs.at[j], gsems[j]).wait()

            def fire_scatter(r, h, j):
                pltpu.async_copy(
                    bufs.at[j], acc_sh.at[didx(r, h)], ssems[j], add=True)

            def wait_scatter(r, h, j):
                pltpu.make_async_copy(
                    bufs.at[j], acc_sh.at[didx(r, h)], ssems[j]).wait()

            BPR = 128 // BATCH
            RPG = NBUF // BPR
            L = NBUF // 2
            nq = ROWS_SC // RPG

            def group(q, carry):
                for m in range(NBUF):
                    r = RPG * q + m // BPR
                    h = m % BPR
                    wait_gather(r, h, m)
                    fire_scatter(r, h, m)
                    jn = (m + L) % NBUF
                    r2 = RPG * q + (m - L) // BPR
                    h2 = (m - L) % BPR
                    rn = RPG * q + (m + L) // BPR
                    hn = (m + L) % BPR
                    if m < L:
                        @pl.when(q > 0)
                        def _():
                            wait_scatter(r2, h2, jn)
                        fire_gather(rn, hn, jn)
                    else:
                        wait_scatter(r2, h2, jn)

                        @pl.when(q < nq - 1)
                        def _():
                            fire_gather(rn, hn, jn)
                return carry

            def chunk_body(ci, carry):
                rbase = rb + ci * ROWS_SC
                cp1 = pltpu.async_copy(
                    src_hbm.at[pl.ds(rbase, ROWS_SC)], src_v, gsems[0])
                cp2 = pltpu.async_copy(
                    dst_hbm.at[pl.ds(rbase, ROWS_SC)], dst_v, gsems[1])
                cp1.wait()
                cp2.wait()
                for b in range(L):
                    fire_gather(b // BPR, b % BPR, b)
                lax.fori_loop(0, nq, group, 0)
                nb = ROWS_SC * BPR
                for b in range(nb - L, nb):
                    wait_scatter(b // BPR, b % BPR, b % NBUF)
                return carry

            lax.fori_loop(0, nsc, chunk_body, 0)

            plsc.subcore_barrier()
            for k in range(ROWS_OUT // 128):
                pltpu.sync_copy(acc_sh.at[pl.ds(ob + k * 128, 128)],
                                otab.at[pl.ds(ob + k * 128, 128)])

        @pl.when(c == 0)
        def _():
            body(ytab, o0)

        @pl.when(c == 1)
        def _():
            body(ytab, o1)

    return scat


_scatter = _make_scatter()


_BLK = 1024
_GRID = N_PAD // _BLK


def _dis_block(pt):
    deg = jnp.sum(pt, axis=1, keepdims=True) + 1.0
    return lax.rsqrt(deg)


def _tca_body(x_ref, pt_ref, xs_ref):
    xs_ref[...] = x_ref[...] * _dis_block(pt_ref[...])


def _tcb_body(a0_ref, a1_ref, xs_ref, pt_ref, w1_ref, b1_ref, w2_ref,
              y2_ref):
    dis = _dis_block(pt_ref[...])
    mx = (a0_ref[...] + a1_ref[...] + xs_ref[...]) * dis
    h = jnp.maximum(
        jnp.dot(mx, w1_ref[...], preferred_element_type=jnp.float32)
        + b1_ref[...], 0.0)
    y2_ref[...] = jnp.dot(h, w2_ref[...],
                          preferred_element_type=jnp.float32) * dis


def _tcc_body(a0_ref, a1_ref, y2_ref, pt_ref, b2_ref, out_ref):
    dis = _dis_block(pt_ref[...])
    pre = a0_ref[...] + a1_ref[...] + y2_ref[...]
    out_ref[...] = jnp.maximum(pre * dis + b2_ref[...], 0.0)


def _row_spec(d):
    return pl.BlockSpec((_BLK, d), lambda i: (i, 0))


def _full_spec(r, d):
    return pl.BlockSpec((r, d), lambda i: (0, 0))


_tca = pl.pallas_call(
    _tca_body,
    grid=(_GRID,),
    in_specs=[_row_spec(D_IN), _row_spec(NC * NS)],
    out_specs=_row_spec(D_IN),
    out_shape=jax.ShapeDtypeStruct((N_PAD, D_IN), jnp.float32),
)

_tcb = pl.pallas_call(
    _tcb_body,
    grid=(_GRID,),
    in_specs=[_row_spec(D_IN), _row_spec(D_IN), _row_spec(D_IN),
              _row_spec(NC * NS), _full_spec(D_IN, D_H), _full_spec(1, D_H),
              _full_spec(D_H, D_OUT)],
    out_specs=_row_spec(D_OUT),
    out_shape=jax.ShapeDtypeStruct((N_PAD, D_OUT), jnp.float32),
)

_tcc = pl.pallas_call(
    _tcc_body,
    grid=(_GRID,),
    in_specs=[_row_spec(D_OUT), _row_spec(D_OUT), _row_spec(D_OUT),
              _row_spec(NC * NS), _full_spec(1, D_OUT)],
    out_specs=_row_spec(D_OUT),
    out_shape=jax.ShapeDtypeStruct((N_PAD, D_OUT), jnp.float32),
)


def kernel(x, edge_index, W1, b1, W2, b2):
    ei = edge_index.astype(jnp.int32)
    pad = PAD_NODE + (jnp.arange(E_PAD - E, dtype=jnp.int32) % (N_PAD - N))
    src2d = jnp.concatenate([ei[0], pad]).reshape(E_ROWS, 128)
    dst2d = jnp.concatenate([ei[1], pad]).reshape(E_ROWS, 128)
    x_pad = jnp.pad(x, ((0, N_PAD - N), (0, 0)))

    partials = _deg_kernel(dst2d)
    pt = partials.T

    xs = _tca(x_pad, pt)
    a1_0, a1_1 = _scatter(xs, src2d, dst2d)
    y2 = _tcb(a1_0, a1_1, xs, pt, W1, b1.reshape(1, D_H), W2)
    a2_0, a2_1 = _scatter(y2, src2d, dst2d)
    out = _tcc(a2_0, a2_1, y2, pt, b2.reshape(1, D_OUT))
    return out[:N]


# speedup vs baseline: 36.0473x; 1.0209x over previous
import functools

import jax
import jax.numpy as jnp
from jax import lax
from jax.experimental import pallas as pl
from jax.experimental.pallas import tpu as pltpu
from jax.experimental.pallas import tpu_sc as plsc

N = 10000
D_IN = 128
D_H = 256
D_OUT = 128
E = 320000

N_PAD = 10240
PAD_NODE = N
E_ROWS = 2560
E_PAD = E_ROWS * 128

NC = 2
NS = 16
ROWS_W = E_ROWS // (NC * NS)
ROWS_T = E_ROWS // NS
ROWS_SC = 40
BATCH = 32
NBUF = 8
NSC = ROWS_T // ROWS_SC
ROWS_OUT = N_PAD // NS

_mesh = plsc.VectorSubcoreMesh(core_axis_name="c", subcore_axis_name="s")
_sc_params = pltpu.CompilerParams(needs_layout_passes=False)


@functools.partial(
    pl.kernel,
    out_type=jax.ShapeDtypeStruct((NC * NS, N_PAD), jnp.float32),
    mesh=_mesh,
    compiler_params=_sc_params,
    scratch_types=[
        pltpu.VMEM((ROWS_W, 128), jnp.int32),
        pltpu.VMEM((N_PAD,), jnp.float32),
    ],
)
def _deg_kernel(dst_hbm, out_hbm, idx_v, hist_v):
    c = lax.axis_index("c")
    s = lax.axis_index("s")
    w = c * NS + s

    zero16 = jnp.zeros((16,), jnp.float32)

    def zbody(i, carry):
        hist_v[pl.ds(i * 16, 16)] = zero16
        return carry

    lax.fori_loop(0, N_PAD // 16, zbody, 0)

    pltpu.sync_copy(dst_hbm.at[pl.ds(w * ROWS_W, ROWS_W)], idx_v)

    ones16 = jnp.ones((16,), jnp.float32)

    def ebody(i, carry):
        r = i // 8
        j = i % 8
        iv = idx_v[r, pl.ds(j * 16, 16)]
        plsc.addupdate_scatter(hist_v, [iv], ones16)
        return carry

    lax.fori_loop(0, ROWS_W * 8, ebody, 0)

    pltpu.sync_copy(hist_v, out_hbm.at[w])


def _make_scatter():
    D = 128
    rows_t = E_ROWS // (NC * NS)
    nsc = rows_t // ROWS_SC

    @functools.partial(
        pl.kernel,
        out_type=(
            jax.ShapeDtypeStruct((N_PAD, D), jnp.float32),
            jax.ShapeDtypeStruct((N_PAD, D), jnp.float32),
        ),
        mesh=_mesh,
        compiler_params=_sc_params,
        scratch_types=[
            pltpu.VMEM((ROWS_SC, 128), jnp.int32),
            pltpu.VMEM((ROWS_SC, 128), jnp.int32),
            pltpu.VMEM((NBUF, BATCH, D), jnp.float32),
            pltpu.VMEM_SHARED((N_PAD, D), jnp.float32),
        ] + [pltpu.SemaphoreType.DMA] * (2 * NBUF),
    )
    def scat(ytab, src_hbm, dst_hbm, o0, o1,
             src_v, dst_v, bufs, acc_sh, *sems):
        c = lax.axis_index("c")
        s = lax.axis_index("s")
        gsems = sems[:NBUF]
        ssems = sems[NBUF:]

        rb = (c * NS + s) * rows_t

        zero16 = jnp.zeros((16,), jnp.float32)

        def zbody(i, carry):
            r = i // (D // 16)
            j = i % (D // 16)
            bufs[0, r, pl.ds(j * 16, 16)] = zero16
            return carry

        lax.fori_loop(0, BATCH * (D // 16), zbody, 0)
        ob = s * ROWS_OUT
        for k in range(ROWS_OUT // BATCH):
            pltpu.sync_copy(bufs.at[0],
                            acc_sh.at[pl.ds(ob + k * BATCH, BATCH)])
        plsc.subcore_barrier()

        def body(ytab, otab):
            def gidx(r, h):
                return src_v.at[r, pl.ds(h * BATCH, BATCH)]

            def didx(r, h):
                return dst_v.at[r, pl.ds(h * BATCH, BATCH)]

            def fire_gather(r, h, j):
                pltpu.async_copy(ytab.at[gidx(r, h)], bufs.at[j], gsems[j])

            def wait_gather(r, h, j):
                pltpu.make_async_copy(
                    ytab.at[gidx(r, h)], bufs.at[j], gsems[j]).wait()

            def fire_scatter(r, h, j):
                pltpu.async_copy(
                    bufs.at[j], acc_sh.at[didx(r, h)], ssems[j], add=True)

            def wait_scatter(r, h, j):
                pltpu.make_async_copy(
                    bufs.at[j], acc_sh.at[didx(r, h)], ssems[j]).wait()

            BPR = 128 // BATCH
            RPG = NBUF // BPR
            L = NBUF // 2
            nq = ROWS_SC // RPG

            def group(q, carry):
                for m in range(NBUF):
                    r = RPG * q + m // BPR
                    h = m % BPR
                    wait_gather(r, h, m)
                    fire_scatter(r, h, m)
                    jn = (m + L) % NBUF
                    r2 = RPG * q + (m - L) // BPR
                    h2 = (m - L) % BPR
                    rn = RPG * q + (m + L) // BPR
                    hn = (m + L) % BPR
                    if m < L:
                        @pl.when(q > 0)
                        def _():
                            wait_scatter(r2, h2, jn)
                        fire_gather(rn, hn, jn)
                    else:
                        wait_scatter(r2, h2, jn)

                        @pl.when(q < nq - 1)
                        def _():
                            fire_gather(rn, hn, jn)
                return carry

            def chunk_body(ci, carry):
                rbase = rb + ci * ROWS_SC
                cp1 = pltpu.async_copy(
                    src_hbm.at[pl.ds(rbase, ROWS_SC)], src_v, gsems[0])
                cp2 = pltpu.async_copy(
                    dst_hbm.at[pl.ds(rbase, ROWS_SC)], dst_v, gsems[1])
                cp1.wait()
                cp2.wait()
                for b in range(L):
                    fire_gather(b // BPR, b % BPR, b)
                lax.fori_loop(0, nq, group, 0)
                nb = ROWS_SC * BPR
                for b in range(nb - L, nb):
                    wait_scatter(b // BPR, b % BPR, b % NBUF)
                return carry

            lax.fori_loop(0, nsc, chunk_body, 0)

            plsc.subcore_barrier()
            for k in range(ROWS_OUT // 128):
                pltpu.sync_copy(acc_sh.at[pl.ds(ob + k * 128, 128)],
                                otab.at[pl.ds(ob + k * 128, 128)])

        @pl.when(c == 0)
        def _():
            body(ytab, o0)

        @pl.when(c == 1)
        def _():
            body(ytab, o1)

    return scat


_scatter = _make_scatter()


_BLK = 2048
_GRID = N_PAD // _BLK


def _dis_block(pt):
    deg = jnp.sum(pt, axis=1, keepdims=True) + 1.0
    return lax.rsqrt(deg)


def _tca_body(x_ref, pt_ref, xs_ref):
    xs_ref[...] = x_ref[...] * _dis_block(pt_ref[...])


def _tcb_body(a0_ref, a1_ref, xs_ref, pt_ref, w1_ref, b1_ref, w2_ref,
              y2_ref):
    dis = _dis_block(pt_ref[...])
    mx = (a0_ref[...] + a1_ref[...] + xs_ref[...]) * dis
    h = jnp.maximum(
        jnp.dot(mx, w1_ref[...], preferred_element_type=jnp.float32)
        + b1_ref[...], 0.0)
    y2_ref[...] = jnp.dot(h, w2_ref[...],
                          preferred_element_type=jnp.float32) * dis


def _tcc_body(a0_ref, a1_ref, y2_ref, pt_ref, b2_ref, out_ref):
    dis = _dis_block(pt_ref[...])
    pre = a0_ref[...] + a1_ref[...] + y2_ref[...]
    out_ref[...] = jnp.maximum(pre * dis + b2_ref[...], 0.0)


def _row_spec(d):
    return pl.BlockSpec((_BLK, d), lambda i: (i, 0))


def _full_spec(r, d):
    return pl.BlockSpec((r, d), lambda i: (0, 0))


_tca = pl.pallas_call(
    _tca_body,
    grid=(_GRID,),
    in_specs=[_row_spec(D_IN), _row_spec(NC * NS)],
    out_specs=_row_spec(D_IN),
    out_shape=jax.ShapeDtypeStruct((N_PAD, D_IN), jnp.float32),
)

_tcb = pl.pallas_call(
    _tcb_body,
    grid=(_GRID,),
    in_specs=[_row_spec(D_IN), _row_spec(D_IN), _row_spec(D_IN),
              _row_spec(NC * NS), _full_spec(D_IN, D_H), _full_spec(1, D_H),
              _full_spec(D_H, D_OUT)],
    out_specs=_row_spec(D_OUT),
    out_shape=jax.ShapeDtypeStruct((N_PAD, D_OUT), jnp.float32),
)

_tcc = pl.pallas_call(
    _tcc_body,
    grid=(_GRID,),
    in_specs=[_row_spec(D_OUT), _row_spec(D_OUT), _row_spec(D_OUT),
              _row_spec(NC * NS), _full_spec(1, D_OUT)],
    out_specs=_row_spec(D_OUT),
    out_shape=jax.ShapeDtypeStruct((N_PAD, D_OUT), jnp.float32),
)


def kernel(x, edge_index, W1, b1, W2, b2):
    ei = edge_index.astype(jnp.int32)
    pad = PAD_NODE + (jnp.arange(E_PAD - E, dtype=jnp.int32) % (N_PAD - N))
    src2d = jnp.concatenate([ei[0], pad]).reshape(E_ROWS, 128)
    dst2d = jnp.concatenate([ei[1], pad]).reshape(E_ROWS, 128)
    x_pad = jnp.pad(x, ((0, N_PAD - N), (0, 0)))

    partials = _deg_kernel(dst2d)
    pt = partials.T

    xs = _tca(x_pad, pt)
    a1_0, a1_1 = _scatter(xs, src2d, dst2d)
    y2 = _tcb(a1_0, a1_1, xs, pt, W1, b1.reshape(1, D_H), W2)
    a2_0, a2_1 = _scatter(y2, src2d, dst2d)
    out = _tcc(a2_0, a2_1, y2, pt, b2.reshape(1, D_OUT))
    return out[:N]


# speedup vs baseline: 36.0492x; 1.0001x over previous
import functools

import jax
import jax.numpy as jnp
from jax import lax
from jax.experimental import pallas as pl
from jax.experimental.pallas import tpu as pltpu
from jax.experimental.pallas import tpu_sc as plsc

N = 10000
D_IN = 128
D_H = 256
D_OUT = 128
E = 320000

N_PAD = 10240
PAD_NODE = N
E_ROWS = 2560
E_PAD = E_ROWS * 128

NC = 2
NS = 16
ROWS_W = E_ROWS // (NC * NS)
ROWS_T = E_ROWS // NS
ROWS_SC = 40
BATCH = 32
NBUF = 8
NSC = ROWS_T // ROWS_SC
ROWS_OUT = N_PAD // NS

_mesh = plsc.VectorSubcoreMesh(core_axis_name="c", subcore_axis_name="s")
_sc_params = pltpu.CompilerParams(needs_layout_passes=False)


@functools.partial(
    pl.kernel,
    out_type=jax.ShapeDtypeStruct((NC * NS, N_PAD), jnp.float32),
    mesh=_mesh,
    compiler_params=_sc_params,
    scratch_types=[
        pltpu.VMEM((ROWS_W, 128), jnp.int32),
        pltpu.VMEM((N_PAD,), jnp.float32),
    ],
)
def _deg_kernel(dst_hbm, out_hbm, idx_v, hist_v):
    c = lax.axis_index("c")
    s = lax.axis_index("s")
    w = c * NS + s

    zero16 = jnp.zeros((16,), jnp.float32)

    def zbody(i, carry):
        hist_v[pl.ds(i * 16, 16)] = zero16
        return carry

    lax.fori_loop(0, N_PAD // 16, zbody, 0)

    pltpu.sync_copy(dst_hbm.at[pl.ds(w * ROWS_W, ROWS_W)], idx_v)

    ones16 = jnp.ones((16,), jnp.float32)

    def ebody(i, carry):
        r = i // 8
        j = i % 8
        iv = idx_v[r, pl.ds(j * 16, 16)]
        plsc.addupdate_scatter(hist_v, [iv], ones16)
        return carry

    lax.fori_loop(0, ROWS_W * 8, ebody, 0)

    pltpu.sync_copy(hist_v, out_hbm.at[w])


def _make_scatter():
    D = 128
    rows_t = E_ROWS // (NC * NS)
    nsc = rows_t // ROWS_SC

    @functools.partial(
        pl.kernel,
        out_type=(
            jax.ShapeDtypeStruct((N_PAD, D), jnp.float32),
            jax.ShapeDtypeStruct((N_PAD, D), jnp.float32),
        ),
        mesh=_mesh,
        compiler_params=_sc_params,
        scratch_types=[
            pltpu.VMEM((ROWS_SC, 128), jnp.int32),
            pltpu.VMEM((ROWS_SC, 128), jnp.int32),
            pltpu.VMEM((NBUF, BATCH, D), jnp.float32),
            pltpu.VMEM_SHARED((N_PAD, D), jnp.float32),
        ] + [pltpu.SemaphoreType.DMA] * (2 * NBUF),
    )
    def scat(ytab, src_hbm, dst_hbm, o0, o1,
             src_v, dst_v, bufs, acc_sh, *sems):
        c = lax.axis_index("c")
        s = lax.axis_index("s")
        gsems = sems[:NBUF]
        ssems = sems[NBUF:]

        rb = (c * NS + s) * rows_t

        zero16 = jnp.zeros((16,), jnp.float32)

        def zbody(i, carry):
            r = i // (D // 16)
            j = i % (D // 16)
            bufs[0, r, pl.ds(j * 16, 16)] = zero16
            return carry

        lax.fori_loop(0, BATCH * (D // 16), zbody, 0)
        ob = s * ROWS_OUT
        for k in range(ROWS_OUT // BATCH):
            pltpu.sync_copy(bufs.at[0],
                            acc_sh.at[pl.ds(ob + k * BATCH, BATCH)])
        plsc.subcore_barrier()

        def body(ytab, otab):
            def gidx(r, h):
                return src_v.at[r, pl.ds(h * BATCH, BATCH)]

            def didx(r, h):
                return dst_v.at[r, pl.ds(h * BATCH, BATCH)]

            def fire_gather(r, h, j):
                pltpu.async_copy(ytab.at[gidx(r, h)], bufs.at[j], gsems[j])

            def wait_gather(r, h, j):
                pltpu.make_async_copy(
                    ytab.at[gidx(r, h)], bufs.at[j], gsems[j]).wait()

            def fire_scatter(r, h, j):
                pltpu.async_copy(
                    bufs.at[j], acc_sh.at[didx(r, h)], ssems[j], add=True)

            def wait_scatter(r, h, j):
                pltpu.make_async_copy(
                    bufs.at[j], acc_sh.at[didx(r, h)], ssems[j]).wait()

            BPR = 128 // BATCH
            RPG = NBUF // BPR
            L = NBUF // 2
            nq = ROWS_SC // RPG

            def group(q, carry):
                for m in range(NBUF):
                    r = RPG * q + m // BPR
                    h = m % BPR
                    wait_gather(r, h, m)
                    fire_scatter(r, h, m)
                    jn = (m + L) % NBUF
                    r2 = RPG * q + (m - L) // BPR
                    h2 = (m - L) % BPR
                    rn = RPG * q + (m + L) // BPR
                    hn = (m + L) % BPR
                    if m < L:
                        @pl.when(q > 0)
                        def _():
                            wait_scatter(r2, h2, jn)
                        fire_gather(rn, hn, jn)
                    else:
                        wait_scatter(r2, h2, jn)

                        @pl.when(q < nq - 1)
                        def _():
                            fire_gather(rn, hn, jn)
                return carry

            def chunk_body(ci, carry):
                rbase = rb + ci * ROWS_SC
                cp1 = pltpu.async_copy(
                    src_hbm.at[pl.ds(rbase, ROWS_SC)], src_v, gsems[0])
                cp2 = pltpu.async_copy(
                    dst_hbm.at[pl.ds(rbase, ROWS_SC)], dst_v, gsems[1])
                cp1.wait()
                cp2.wait()
                for b in range(L):
                    fire_gather(b // BPR, b % BPR, b)
                lax.fori_loop(0, nq, group, 0)
                nb = ROWS_SC * BPR
                for b in range(nb - L, nb):
                    wait_scatter(b // BPR, b % BPR, b % NBUF)
                return carry

            lax.fori_loop(0, nsc, chunk_body, 0)

            plsc.subcore_barrier()
            for k in range(ROWS_OUT // 128):
                pltpu.sync_copy(acc_sh.at[pl.ds(ob + k * 128, 128)],
                                otab.at[pl.ds(ob + k * 128, 128)])

        @pl.when(c == 0)
        def _():
            body(ytab, o0)

        @pl.when(c == 1)
        def _():
            body(ytab, o1)

    return scat


_scatter = _make_scatter()


_BLK = 10240
_GRID = N_PAD // _BLK


def _dis_block(pt):
    deg = jnp.sum(pt, axis=1, keepdims=True) + 1.0
    return lax.rsqrt(deg)


def _tca_body(x_ref, pt_ref, xs_ref):
    xs_ref[...] = x_ref[...] * _dis_block(pt_ref[...])


def _tcb_body(a0_ref, a1_ref, xs_ref, pt_ref, w1_ref, b1_ref, w2_ref,
              y2_ref):
    dis = _dis_block(pt_ref[...])
    mx = (a0_ref[...] + a1_ref[...] + xs_ref[...]) * dis
    h = jnp.maximum(
        jnp.dot(mx, w1_ref[...], preferred_element_type=jnp.float32)
        + b1_ref[...], 0.0)
    y2_ref[...] = jnp.dot(h, w2_ref[...],
                          preferred_element_type=jnp.float32) * dis


def _tcc_body(a0_ref, a1_ref, y2_ref, pt_ref, b2_ref, out_ref):
    dis = _dis_block(pt_ref[...])
    pre = a0_ref[...] + a1_ref[...] + y2_ref[...]
    out_ref[...] = jnp.maximum(pre * dis + b2_ref[...], 0.0)


def _row_spec(d):
    return pl.BlockSpec((_BLK, d), lambda i: (i, 0))


def _full_spec(r, d):
    return pl.BlockSpec((r, d), lambda i: (0, 0))


_tca = pl.pallas_call(
    _tca_body,
    grid=(_GRID,),
    in_specs=[_row_spec(D_IN), _row_spec(NC * NS)],
    out_specs=_row_spec(D_IN),
    out_shape=jax.ShapeDtypeStruct((N_PAD, D_IN), jnp.float32),
)

_tcb = pl.pallas_call(
    _tcb_body,
    grid=(_GRID,),
    in_specs=[_row_spec(D_IN), _row_spec(D_IN), _row_spec(D_IN),
              _row_spec(NC * NS), _full_spec(D_IN, D_H), _full_spec(1, D_H),
              _full_spec(D_H, D_OUT)],
    out_specs=_row_spec(D_OUT),
    out_shape=jax.ShapeDtypeStruct((N_PAD, D_OUT), jnp.float32),
)

_tcc = pl.pallas_call(
    _tcc_body,
    grid=(_GRID,),
    in_specs=[_row_spec(D_OUT), _row_spec(D_OUT), _row_spec(D_OUT),
              _row_spec(NC * NS), _full_spec(1, D_OUT)],
    out_specs=_row_spec(D_OUT),
    out_shape=jax.ShapeDtypeStruct((N_PAD, D_OUT), jnp.float32),
)


def kernel(x, edge_index, W1, b1, W2, b2):
    ei = edge_index.astype(jnp.int32)
    pad = PAD_NODE + (jnp.arange(E_PAD - E, dtype=jnp.int32) % (N_PAD - N))
    src2d = jnp.concatenate([ei[0], pad]).reshape(E_ROWS, 128)
    dst2d = jnp.concatenate([ei[1], pad]).reshape(E_ROWS, 128)
    x_pad = jnp.pad(x, ((0, N_PAD - N), (0, 0)))

    partials = _deg_kernel(dst2d)
    pt = partials.T

    xs = _tca(x_pad, pt)
    a1_0, a1_1 = _scatter(xs, src2d, dst2d)
    y2 = _tcb(a1_0, a1_1, xs, pt, W1, b1.reshape(1, D_H), W2)
    a2_0, a2_1 = _scatter(y2, src2d, dst2d)
    out = _tcc(a2_0, a2_1, y2, pt, b2.reshape(1, D_OUT))
    return out[:N]
